# baseline (speedup 1.0000x reference)
import functools

import jax
import jax.numpy as jnp
from jax.experimental import pallas as pl
from jax.experimental.pallas import tpu as pltpu


def _dma_body(ptr_ref, keys, queue, out, sem, *, blk, n_rows, k_rows, grid):
    for phase in ("start", "wait"):
        for i in range(grid):
            s = pl.multiple_of((i * blk - ptr_ref[0]) % k_rows, blk)
            take_keys = s < n_rows

            @pl.when(take_keys)
            def _(i=i, s=s):
                cp = pltpu.make_async_copy(
                    keys.at[pl.ds(s, blk)],
                    out.at[pl.ds(i * blk, blk)],
                    sem.at[i],
                )
                cp.start() if phase == "start" else cp.wait()

            @pl.when(jnp.logical_not(take_keys))
            def _(i=i):
                cp = pltpu.make_async_copy(
                    queue.at[pl.ds(i * blk, blk)],
                    out.at[pl.ds(i * blk, blk)],
                    sem.at[i],
                )
                cp.start() if phase == "start" else cp.wait()


def kernel(keys, queue, ptr):
    n, d = keys.shape
    k = queue.shape[0]
    blk = 4096
    grid = k // blk
    ptr_arr = jnp.asarray(ptr, jnp.int32).reshape((1,))
    return pl.pallas_call(
        functools.partial(_dma_body, blk=blk, n_rows=n, k_rows=k, grid=grid),
        in_specs=[
            pl.BlockSpec(memory_space=pltpu.SMEM),
            pl.BlockSpec(memory_space=pl.ANY),
            pl.BlockSpec(memory_space=pl.ANY),
        ],
        out_specs=pl.BlockSpec(memory_space=pl.ANY),
        out_shape=jax.ShapeDtypeStruct((k, d), queue.dtype),
        scratch_shapes=[pltpu.SemaphoreType.DMA((grid,))],
    )(ptr_arr, keys, queue)


# speedup vs baseline: 43.4729x; 43.4729x over previous
import functools

import jax
import jax.numpy as jnp
from jax.experimental import pallas as pl
from jax.experimental.pallas import tpu as pltpu


def _body(ptr_ref, keys_ref, queue_hbm, out_ref):
    out_ref[...] = keys_ref[...]


def kernel(keys, queue, ptr):
    n, d = keys.shape
    k = queue.shape[0]
    blk = n
    ptr_arr = jnp.asarray(ptr, jnp.int32).reshape((1,))

    def keys_map(i, ptr_ref):
        return (0, 0)

    def out_map(i, ptr_ref):
        return ((ptr_ref[0] % k) // blk, 0)

    grid_spec = pltpu.PrefetchScalarGridSpec(
        num_scalar_prefetch=1,
        grid=(1,),
        in_specs=[
            pl.BlockSpec((blk, d), keys_map),
            pl.BlockSpec(memory_space=pl.ANY),
        ],
        out_specs=pl.BlockSpec((blk, d), out_map),
    )
    return pl.pallas_call(
        _body,
        grid_spec=grid_spec,
        out_shape=jax.ShapeDtypeStruct((k, d), queue.dtype),
        input_output_aliases={2: 0},
    )(ptr_arr, keys, queue)
